# baseline (speedup 1.0000x reference)
import functools

import jax
import jax.numpy as jnp
from jax import lax
from jax.experimental import pallas as pl
from jax.experimental.pallas import tpu as pltpu
from jax.experimental.pallas import tpu_sc as plsc

_NUM_EMBEDDINGS = 1024
_DIM = 64
_COMMITMENT = 0.25
_TILE = 2048
_ROWS = _TILE // 1024

_SC_CORES = 2
_SC_SUBCORES = 16
_SC_WORKERS = _SC_CORES * _SC_SUBCORES
_SC_WINDOW = 128


def _dist_tile_kernel(x_ref, e_ref, idx_ref, loss_ref, epad_ref, es_ref, e2_ref):
    e = e_ref[...]

    @pl.when(pl.program_id(0) == 0)
    def _prep():
        es_ref[...] = e * -2.0
        e2_ref[...] = jnp.sum(e * e, axis=1, keepdims=True)
        epad_ref[:, :_DIM] = e
        epad_ref[:, _DIM:] = jnp.zeros((_NUM_EMBEDDINGS, _DIM), jnp.float32)

    x = x_ref[...].reshape(_TILE, _DIM)
    x2 = jnp.sum(x * x, axis=1, keepdims=True)
    one11 = jnp.ones((1, 1), jnp.float32)
    x2t = jax.lax.dot_general(one11, x2, (((1,), (1,)), ((), ())),
                              preferred_element_type=jnp.float32)
    m2t = jax.lax.dot_general(es_ref[...], x, (((1,), (1,)), ((), ())),
                              preferred_element_type=jnp.float32)
    d = (x2t + e2_ref[...]) + m2t

    dmin = jnp.min(d, axis=0, keepdims=True)
    ids = jax.lax.broadcasted_iota(jnp.int32, d.shape, 0)
    idx = jnp.min(jnp.where(d == dmin, ids, _NUM_EMBEDDINGS),
                  axis=0, keepdims=True)

    idx_ref[...] = idx.reshape(_TILE)

    part = jnp.sum(dmin)

    @pl.when(pl.program_id(0) == 0)
    def _init():
        loss_ref[0, 0] = 0.0

    loss_ref[0, 0] += part


def _compact_kernel(qpad_ref, q_ref):
    q_ref[...] = qpad_ref[:, :_DIM].reshape(_ROWS, 1024, _DIM)


def _sc_gather(embedding, idx, tokens):
    b_per_w = tokens // _SC_WORKERS
    n_win = b_per_w // _SC_WINDOW
    mesh = plsc.VectorSubcoreMesh(core_axis_name="c", subcore_axis_name="s")

    @functools.partial(
        pl.kernel, mesh=mesh,
        out_type=jax.ShapeDtypeStruct((tokens, 2 * _DIM), jnp.float32),
        scratch_types=[
            pltpu.VMEM((b_per_w,), jnp.int32),
            pltpu.VMEM((_SC_WINDOW, 2 * _DIM), jnp.float32),
            pltpu.VMEM((_SC_WINDOW, 2 * _DIM), jnp.float32),
            pltpu.SemaphoreType.DMA,
            pltpu.SemaphoreType.DMA,
        ],
    )
    def gather_kernel(table_hbm, idx_hbm, out_hbm, idx_v, rows_a, rows_b, sem_a, sem_b):
        wid = lax.axis_index("s") * _SC_CORES + lax.axis_index("c")
        base = wid * b_per_w
        pltpu.sync_copy(idx_hbm.at[pl.ds(base, b_per_w)], idx_v)
        rows = (rows_a, rows_b)
        sems = (sem_a, sem_b)
        copies = [None, None]

        def _writeback(j, s):
            pltpu.sync_copy(rows[s],
                            out_hbm.at[pl.ds(base + j * _SC_WINDOW,
                                             _SC_WINDOW)])

        for j in range(n_win):
            s = j % 2
            copies[s] = pltpu.async_copy(
                table_hbm.at[idx_v.at[pl.ds(j * _SC_WINDOW, _SC_WINDOW)]],
                rows[s], sems[s])
            if j > 0:
                copies[1 - s].wait()
                _writeback(j - 1, 1 - s)
        copies[(n_win - 1) % 2].wait()
        _writeback(n_win - 1, (n_win - 1) % 2)

    return gather_kernel(embedding, idx)


def kernel(x, embedding):
    tokens = x.shape[0] * x.shape[1]
    grid = tokens // _TILE

    idx, loss_sum, epad = pl.pallas_call(
        _dist_tile_kernel,
        grid=(grid,),
        in_specs=[
            pl.BlockSpec((_ROWS, 1024, _DIM), lambda i: (i, 0, 0)),
            pl.BlockSpec((_NUM_EMBEDDINGS, _DIM), lambda i: (0, 0)),
        ],
        out_specs=[
            pl.BlockSpec((_TILE,), lambda i: (i,)),
            pl.BlockSpec(memory_space=pltpu.SMEM, block_shape=(1, 1),
                         index_map=lambda i: (0, 0)),
            pl.BlockSpec((_NUM_EMBEDDINGS, 2 * _DIM), lambda i: (0, 0)),
        ],
        out_shape=[
            jax.ShapeDtypeStruct((tokens,), jnp.int32),
            jax.ShapeDtypeStruct((1, 1), jnp.float32),
            jax.ShapeDtypeStruct((_NUM_EMBEDDINGS, 2 * _DIM), jnp.float32),
        ],
        scratch_shapes=[
            pltpu.VMEM((_NUM_EMBEDDINGS, _DIM), jnp.float32),
            pltpu.VMEM((_NUM_EMBEDDINGS, 1), jnp.float32),
        ],
    )(x, embedding)

    qpad = _sc_gather(epad, idx, tokens)

    q = pl.pallas_call(
        _compact_kernel,
        grid=(grid,),
        in_specs=[pl.BlockSpec((_TILE, 2 * _DIM), lambda i: (i, 0))],
        out_specs=[pl.BlockSpec((_ROWS, 1024, _DIM), lambda i: (i, 0, 0))],
        out_shape=[jax.ShapeDtypeStruct(x.shape, jnp.float32)],
    )(qpad)[0]

    mean_sq = loss_sum[0, 0] / (tokens * _DIM)
    loss = mean_sq + _COMMITMENT * mean_sq
    return (q, loss, idx)


# speedup vs baseline: 1.2947x; 1.2947x over previous
import jax
import jax.numpy as jnp
from jax.experimental import pallas as pl
from jax.experimental.pallas import tpu as pltpu

_NUM_EMBEDDINGS = 1024
_DIM = 64
_COMMITMENT = 0.25
_TILE = 2048
_ROWS = _TILE // 1024


def _vq_tile_kernel(x_ref, e_ref, q_ref, idx_ref, loss_ref, es_ref, e2_ref):
    e = e_ref[...]

    @pl.when(pl.program_id(0) == 0)
    def _prep():
        es_ref[...] = e * -2.0
        e2_ref[...] = jnp.sum(e * e, axis=1, keepdims=True)

    x = x_ref[...].reshape(_TILE, _DIM)
    x2 = jnp.sum(x * x, axis=1, keepdims=True)
    one11 = jnp.ones((1, 1), jnp.float32)
    x2t = jax.lax.dot_general(one11, x2, (((1,), (1,)), ((), ())),
                              preferred_element_type=jnp.float32)
    m2t = jax.lax.dot_general(es_ref[...], x, (((1,), (1,)), ((), ())),
                              preferred_element_type=jnp.float32)
    d = (x2t + e2_ref[...]) + m2t

    dmin = jnp.min(d, axis=0, keepdims=True)
    ids = jax.lax.broadcasted_iota(jnp.int32, d.shape, 0)
    idx = jnp.min(jnp.where(d == dmin, ids, _NUM_EMBEDDINGS),
                  axis=0, keepdims=True)

    idx_ref[...] = idx.reshape(_TILE)

    onehot_t = (ids == idx).astype(jnp.float32)
    q = jax.lax.dot_general(onehot_t, e, (((0,), (0,)), ((), ())),
                            preferred_element_type=jnp.float32)

    diff = q - x
    part = jnp.sum(diff * diff)

    q_ref[...] = (x + (q - x)).reshape(_ROWS, 1024, _DIM)

    @pl.when(pl.program_id(0) == 0)
    def _init():
        loss_ref[0, 0] = 0.0

    loss_ref[0, 0] += part


def kernel(x, embedding):
    tokens = x.shape[0] * x.shape[1]
    grid = tokens // _TILE

    q, idx, loss_sum = pl.pallas_call(
        _vq_tile_kernel,
        grid=(grid,),
        in_specs=[
            pl.BlockSpec((_ROWS, 1024, _DIM), lambda i: (i, 0, 0)),
            pl.BlockSpec((_NUM_EMBEDDINGS, _DIM), lambda i: (0, 0)),
        ],
        out_specs=[
            pl.BlockSpec((_ROWS, 1024, _DIM), lambda i: (i, 0, 0)),
            pl.BlockSpec((_TILE,), lambda i: (i,)),
            pl.BlockSpec(memory_space=pltpu.SMEM, block_shape=(1, 1),
                         index_map=lambda i: (0, 0)),
        ],
        out_shape=[
            jax.ShapeDtypeStruct(x.shape, jnp.float32),
            jax.ShapeDtypeStruct((tokens,), jnp.int32),
            jax.ShapeDtypeStruct((1, 1), jnp.float32),
        ],
        scratch_shapes=[
            pltpu.VMEM((_NUM_EMBEDDINGS, _DIM), jnp.float32),
            pltpu.VMEM((_NUM_EMBEDDINGS, 1), jnp.float32),
        ],
    )(x, embedding)

    mean_sq = loss_sum[0, 0] / (tokens * _DIM)
    loss = mean_sq + _COMMITMENT * mean_sq
    return (q, loss, idx)
